# baseline (speedup 1.0000x reference)
import functools

import jax
import jax.numpy as jnp
from jax import lax
from jax.experimental import pallas as pl
from jax.experimental.pallas import tpu as pltpu
from jax.experimental.pallas import tpu_sc as plsc

N = 10000
D = 128
NC = 2
NS = 16
NW = NC * NS
CHUNK = 128
N_PAD = 10240
E = 320000
TPC = 80
TPCA = TPC + 2
SEG = N_PAD // NS

@functools.cache
def _sc_kernels():
    mesh = plsc.VectorSubcoreMesh(
        core_axis_name="c", subcore_axis_name="s", num_cores=NC, num_subcores=NS
    )
    deg = pl.kernel(
        _deg_body,
        out_type=jax.ShapeDtypeStruct((NC, 2, N_PAD), jnp.float32),
        mesh=mesh,
        scratch_types=[
            pltpu.VMEM((TPCA, CHUNK), jnp.int32),
            pltpu.VMEM((TPCA, CHUNK), jnp.int32),
            pltpu.VMEM((CHUNK,), jnp.float32),
            pltpu.VMEM_SHARED((N_PAD,), jnp.float32),
            pltpu.VMEM_SHARED((N_PAD,), jnp.float32),
            pltpu.SemaphoreType.DMA,
        ],
    )
    agg = pl.kernel(
        _agg_body,
        out_type=jax.ShapeDtypeStruct((NC, N_PAD, D), jnp.float32),
        mesh=mesh,
        scratch_types=[
            pltpu.VMEM((CHUNK,), jnp.int32),
            pltpu.VMEM((CHUNK,), jnp.int32),
            pltpu.VMEM((TPC, CHUNK), jnp.int32),
            pltpu.VMEM((CHUNK, D), jnp.float32),
            pltpu.VMEM((CHUNK, D), jnp.float32),
            pltpu.VMEM_SHARED((N_PAD, D), jnp.float32),
            pltpu.SemaphoreType.DMA,
            pltpu.SemaphoreType.DMA,
            pltpu.SemaphoreType.DMA,
            pltpu.SemaphoreType.DMA,
            pltpu.SemaphoreType.DMA,
            pltpu.SemaphoreType.DMA,
        ],
    )
    return deg, agg


def _deg_body(src_hbm, dst_hbm, z2_hbm, out_hbm, sidx, didx, ones_v, dout_acc, din_acc, dsem):
    c = lax.axis_index("c")
    s = lax.axis_index("s")
    wid = s * NC + c
    pltpu.sync_copy(z2_hbm.at[0, pl.ds(s * SEG, SEG)], dout_acc.at[pl.ds(s * SEG, SEG)])
    pltpu.sync_copy(z2_hbm.at[1, pl.ds(s * SEG, SEG)], din_acc.at[pl.ds(s * SEG, SEG)])
    pltpu.sync_copy(src_hbm.at[wid], sidx)
    pltpu.sync_copy(dst_hbm.at[wid], didx)

    def _ones(i, carry):
        ones_v[pl.ds(i * 16, 16)] = jnp.ones((16,), jnp.float32)
        return carry

    lax.fori_loop(0, CHUNK // 16, _ones, 0)
    plsc.subcore_barrier()

    def _step(j, carry):
        pltpu.async_copy(ones_v, dout_acc.at[sidx.at[j]], dsem, add=True)
        pltpu.async_copy(ones_v, din_acc.at[didx.at[j]], dsem, add=True)
        return carry

    lax.fori_loop(0, TPC, _step, 0)

    def _drain(j, carry):
        pltpu.make_async_copy(ones_v, dout_acc.at[sidx.at[0]], dsem).wait()
        pltpu.make_async_copy(ones_v, din_acc.at[didx.at[0]], dsem).wait()
        return carry

    lax.fori_loop(0, TPC, _drain, 0)
    plsc.subcore_barrier()
    pltpu.sync_copy(dout_acc.at[pl.ds(s * SEG, SEG)], out_hbm.at[c, 0, pl.ds(s * SEG, SEG)])
    pltpu.sync_copy(din_acc.at[pl.ds(s * SEG, SEG)], out_hbm.at[c, 1, pl.ds(s * SEG, SEG)])


def _agg_body(src_hbm, dst_hbm, hs_hbm, z_hbm, out_hbm,
              sr0, sr1, didx, rows0, rows1, acc, g0, g1, s0, s1, i0, i1):
    c = lax.axis_index("c")
    s = lax.axis_index("s")
    wid = s * NC + c
    pltpu.sync_copy(z_hbm.at[pl.ds(s * SEG, SEG)], acc.at[pl.ds(s * SEG, SEG)])
    pltpu.sync_copy(dst_hbm.at[wid, pl.ds(0, TPC)], didx)
    plsc.subcore_barrier()

    def ifetch(j, buf, sem):
        pltpu.async_copy(src_hbm.at[wid, j], buf, sem)

    def iwait(buf, sem):
        pltpu.make_async_copy(src_hbm.at[wid, 0], buf, sem).wait()

    def gather(sbuf, buf, sem):
        pltpu.async_copy(hs_hbm.at[sbuf], buf, sem)

    def gwait(buf, sem):
        pltpu.make_async_copy(hs_hbm.at[sr0], buf, sem).wait()

    def scat(j, buf, sem):
        pltpu.async_copy(buf, acc.at[didx.at[j]], sem, add=True)

    def swait(buf, sem):
        pltpu.make_async_copy(buf, acc.at[didx.at[0]], sem).wait()

    ifetch(0, sr0, i0)
    ifetch(1, sr1, i1)
    iwait(sr0, i0)
    gather(sr0, rows0, g0)

    def _body(i, carry):
        j0 = 2 * i

        @pl.when(i > 0)
        def _():
            swait(rows1, s1)

        iwait(sr1, i1)
        gather(sr1, rows1, g1)
        gwait(rows0, g0)
        ifetch(j0 + 2, sr0, i0)
        scat(j0, rows0, s0)
        gwait(rows1, g1)
        ifetch(j0 + 3, sr1, i1)
        scat(j0 + 1, rows1, s1)
        swait(rows0, s0)
        iwait(sr0, i0)
        gather(sr0, rows0, g0)
        return carry

    lax.fori_loop(0, TPC // 2, _body, 0)
    gwait(rows0, g0)
    swait(rows1, s1)
    iwait(sr1, i1)
    plsc.subcore_barrier()
    pltpu.sync_copy(acc.at[pl.ds(s * SEG, SEG)], out_hbm.at[c, pl.ds(s * SEG, SEG)])


BLK = 1024
GRID = N_PAD // BLK


def _norm(a, b):
    return lax.rsqrt(jnp.maximum(a + b, 1.0))


def _mm_scale_body(x_ref, w_ref, d0_ref, d1_ref, o_ref):
    ns = _norm(d0_ref[...], d1_ref[...])
    o_ref[...] = jnp.dot(x_ref[...], w_ref[...], preferred_element_type=jnp.float32) * ns


def _post_mm_body(p_ref, w_ref, i0_ref, i1_ref, o0_ref, o1_ref, b_ref, o_ref):
    nin = _norm(i0_ref[...], i1_ref[...])
    h = jnp.maximum((p_ref[0] + p_ref[1]) * nin + b_ref[...], 0.0)
    nout = _norm(o0_ref[...], o1_ref[...])
    o_ref[...] = jnp.dot(h, w_ref[...], preferred_element_type=jnp.float32) * nout


def _post_body(p_ref, i0_ref, i1_ref, b_ref, o_ref):
    nin = _norm(i0_ref[...], i1_ref[...])
    o_ref[...] = jnp.maximum((p_ref[0] + p_ref[1]) * nin + b_ref[...], 0.0)


_col = pl.BlockSpec((BLK, 1), lambda i: (i, 0))
_full = pl.BlockSpec((D, D), lambda i: (0, 0))
_rowblk = pl.BlockSpec((BLK, D), lambda i: (i, 0))
_pblk = pl.BlockSpec((NC, BLK, D), lambda i: (0, i, 0))
_bias = pl.BlockSpec((1, D), lambda i: (0, 0))
_out_t = jax.ShapeDtypeStruct((N_PAD, D), jnp.float32)

_mm_scale = pl.pallas_call(
    _mm_scale_body,
    grid=(GRID,),
    in_specs=[_rowblk, _full, _col, _col],
    out_specs=_rowblk,
    out_shape=_out_t,
)

_post_mm = pl.pallas_call(
    _post_mm_body,
    grid=(GRID,),
    in_specs=[_pblk, _full, _col, _col, _col, _col, _bias],
    out_specs=_rowblk,
    out_shape=_out_t,
)

_post = pl.pallas_call(
    _post_body,
    grid=(GRID,),
    in_specs=[_pblk, _col, _col, _bias],
    out_specs=_rowblk,
    out_shape=_out_t,
)


def kernel(edge_index, x, W1, b1, W2, b2):
    src = edge_index[0]
    dst = edge_index[1]
    pad = N + (lax.iota(jnp.int32, NW * TPC * CHUNK - E) % (N_PAD - N))
    main = jnp.concatenate([src, pad]).reshape(NW, TPC, CHUNK)
    maind = jnp.concatenate([dst, pad]).reshape(NW, TPC, CHUNK)
    padc = (N + (lax.iota(jnp.int32, NW * 2 * CHUNK) % (N_PAD - N))).reshape(NW, 2, CHUNK)
    src3 = jnp.concatenate([main, padc], axis=1)
    dst3 = jnp.concatenate([maind, padc], axis=1)
    xp = jnp.pad(x, ((0, N_PAD - N), (0, 0)))
    z2 = jnp.zeros((2, N_PAD), jnp.float32)
    zN = jnp.zeros((N_PAD, D), jnp.float32)

    _deg, _agg = _sc_kernels()
    degs = _deg(src3, dst3, z2)
    dout0 = degs[0, 0].reshape(N_PAD, 1)
    dout1 = degs[1, 0].reshape(N_PAD, 1)
    din0 = degs[0, 1].reshape(N_PAD, 1)
    din1 = degs[1, 1].reshape(N_PAD, 1)
    b1r = b1.reshape(1, D)
    b2r = b2.reshape(1, D)

    hs1 = _mm_scale(xp, W1, dout0, dout1)
    p1 = _agg(src3, dst3, hs1, zN)
    hs2 = _post_mm(p1, W2, din0, din1, dout0, dout1, b1r)
    p2 = _agg(src3, dst3, hs2, zN)
    out = _post(p2, din0, din1, b2r)
    return out[:N]


# speedup vs baseline: 1.2953x; 1.2953x over previous
import functools

import jax
import jax.numpy as jnp
from jax import lax
from jax.experimental import pallas as pl
from jax.experimental.pallas import tpu as pltpu
from jax.experimental.pallas import tpu_sc as plsc

N = 10000
D = 128
NC = 2
NS = 16
NW = NC * NS
CHUNK = 120
N_PAD = 10240
E = 320000
TPC = 84
TPCA = TPC + 4
SEG = N_PAD // NS

@functools.cache
def _sc_kernels():
    mesh = plsc.VectorSubcoreMesh(
        core_axis_name="c", subcore_axis_name="s", num_cores=NC, num_subcores=NS
    )
    deg = pl.kernel(
        _deg_body,
        out_type=jax.ShapeDtypeStruct((NC, 2, N_PAD), jnp.float32),
        mesh=mesh,
        scratch_types=[
            pltpu.VMEM((TPCA, CHUNK), jnp.int32),
            pltpu.VMEM((TPCA, CHUNK), jnp.int32),
            pltpu.VMEM((CHUNK,), jnp.float32),
            pltpu.VMEM_SHARED((N_PAD,), jnp.float32),
            pltpu.VMEM_SHARED((N_PAD,), jnp.float32),
            pltpu.SemaphoreType.DMA,
        ],
    )
    agg = pl.kernel(
        _agg_body,
        out_type=jax.ShapeDtypeStruct((NC, N_PAD, D), jnp.float32),
        mesh=mesh,
        scratch_types=(
            [pltpu.VMEM((CHUNK,), jnp.int32)] * 6
            + [pltpu.VMEM((CHUNK, D), jnp.float32)] * 3
            + [pltpu.VMEM_SHARED((N_PAD, D), jnp.float32)]
            + [pltpu.SemaphoreType.DMA] * 12
        ),
    )
    return deg, agg


def _deg_body(src_hbm, dst_hbm, z2_hbm, out_hbm, sidx, didx, ones_v, dout_acc, din_acc, dsem):
    c = lax.axis_index("c")
    s = lax.axis_index("s")
    wid = s * NC + c
    pltpu.sync_copy(z2_hbm.at[0, pl.ds(s * SEG, SEG)], dout_acc.at[pl.ds(s * SEG, SEG)])
    pltpu.sync_copy(z2_hbm.at[1, pl.ds(s * SEG, SEG)], din_acc.at[pl.ds(s * SEG, SEG)])
    pltpu.sync_copy(src_hbm.at[wid], sidx)
    pltpu.sync_copy(dst_hbm.at[wid], didx)

    for off in (0, 16, 32, 48, 64, 80, 96, CHUNK - 16):
        ones_v[pl.ds(off, 16)] = jnp.ones((16,), jnp.float32)
    plsc.subcore_barrier()

    def _step(j, carry):
        pltpu.async_copy(ones_v, dout_acc.at[sidx.at[j]], dsem, add=True)
        pltpu.async_copy(ones_v, din_acc.at[didx.at[j]], dsem, add=True)
        return carry

    lax.fori_loop(0, TPC, _step, 0)

    def _drain(j, carry):
        pltpu.make_async_copy(ones_v, dout_acc.at[sidx.at[0]], dsem).wait()
        pltpu.make_async_copy(ones_v, din_acc.at[didx.at[0]], dsem).wait()
        return carry

    lax.fori_loop(0, TPC, _drain, 0)
    plsc.subcore_barrier()
    pltpu.sync_copy(dout_acc.at[pl.ds(s * SEG, SEG)], out_hbm.at[c, 0, pl.ds(s * SEG, SEG)])
    pltpu.sync_copy(din_acc.at[pl.ds(s * SEG, SEG)], out_hbm.at[c, 1, pl.ds(s * SEG, SEG)])


def _agg_body(src_hbm, dst_hbm, hs_hbm, z_hbm, out_hbm,
              sr0, sr1, sr2, dr0, dr1, dr2, rows0, rows1, rows2, acc,
              g0, g1, g2, s0, s1, s2, i0, i1, i2, d0, d1, d2):
    c = lax.axis_index("c")
    s = lax.axis_index("s")
    wid = s * NC + c
    srs = (sr0, sr1, sr2)
    drs = (dr0, dr1, dr2)
    rows = (rows0, rows1, rows2)
    gs = (g0, g1, g2)
    ss = (s0, s1, s2)
    is_ = (i0, i1, i2)
    ds_ = (d0, d1, d2)

    pltpu.sync_copy(z_hbm.at[pl.ds(s * SEG, SEG)], acc.at[pl.ds(s * SEG, SEG)])

    def ifetch(j, b):
        pltpu.async_copy(src_hbm.at[wid, j], srs[b], is_[b])

    def iwait(b):
        pltpu.make_async_copy(src_hbm.at[wid, 0], srs[b], is_[b]).wait()

    def dfetch(j, b):
        pltpu.async_copy(dst_hbm.at[wid, j], drs[b], ds_[b])

    def dwait(b):
        pltpu.make_async_copy(dst_hbm.at[wid, 0], drs[b], ds_[b]).wait()

    def gather(b):
        pltpu.async_copy(hs_hbm.at[srs[b]], rows[b], gs[b])

    def gwait(b):
        pltpu.make_async_copy(hs_hbm.at[sr0], rows[b], gs[b]).wait()

    def scat(b):
        pltpu.async_copy(rows[b], acc.at[drs[b]], ss[b], add=True)

    def swait(b):
        pltpu.make_async_copy(rows[b], acc.at[dr0], ss[b]).wait()

    ifetch(0, 0)
    ifetch(1, 1)
    ifetch(2, 2)
    dfetch(0, 0)
    dfetch(1, 1)
    iwait(0)
    gather(0)
    iwait(1)
    gather(1)
    plsc.subcore_barrier()

    def _stage(j, b, guard):
        cs = (b + 2) % 3
        gwait(b)
        dwait(b)
        scat(b)
        ifetch(j + 3, b)

        if guard is None:
            swait(cs)
        else:
            @pl.when(guard)
            def _():
                swait(cs)

        dfetch(j + 2, cs)
        iwait(cs)
        gather(cs)

    def _body(i, carry):
        j0 = 3 * i
        _stage(j0, 0, i > 0)
        _stage(j0 + 1, 1, None)
        _stage(j0 + 2, 2, None)
        return carry

    lax.fori_loop(0, TPC // 3, _body, 0)
    gwait(0)
    gwait(1)
    swait(2)
    iwait(2)
    dwait(0)
    dwait(1)
    plsc.subcore_barrier()
    pltpu.sync_copy(acc.at[pl.ds(s * SEG, SEG)], out_hbm.at[c, pl.ds(s * SEG, SEG)])


BLK = 1024
GRID = N_PAD // BLK


def _norm(a, b):
    return lax.rsqrt(jnp.maximum(a + b, 1.0))


def _mm_scale_body(x_ref, w_ref, d0_ref, d1_ref, o_ref):
    ns = _norm(d0_ref[...], d1_ref[...])
    o_ref[...] = jnp.dot(x_ref[...], w_ref[...], preferred_element_type=jnp.float32) * ns


def _post_mm_body(p_ref, w_ref, i0_ref, i1_ref, o0_ref, o1_ref, b_ref, o_ref):
    nin = _norm(i0_ref[...], i1_ref[...])
    h = jnp.maximum((p_ref[0] + p_ref[1]) * nin + b_ref[...], 0.0)
    nout = _norm(o0_ref[...], o1_ref[...])
    o_ref[...] = jnp.dot(h, w_ref[...], preferred_element_type=jnp.float32) * nout


def _post_body(p_ref, i0_ref, i1_ref, b_ref, o_ref):
    nin = _norm(i0_ref[...], i1_ref[...])
    o_ref[...] = jnp.maximum((p_ref[0] + p_ref[1]) * nin + b_ref[...], 0.0)


_col = pl.BlockSpec((BLK, 1), lambda i: (i, 0))
_full = pl.BlockSpec((D, D), lambda i: (0, 0))
_rowblk = pl.BlockSpec((BLK, D), lambda i: (i, 0))
_pblk = pl.BlockSpec((NC, BLK, D), lambda i: (0, i, 0))
_bias = pl.BlockSpec((1, D), lambda i: (0, 0))
_out_t = jax.ShapeDtypeStruct((N_PAD, D), jnp.float32)

_mm_scale = pl.pallas_call(
    _mm_scale_body,
    grid=(GRID,),
    in_specs=[_rowblk, _full, _col, _col],
    out_specs=_rowblk,
    out_shape=_out_t,
)

_post_mm = pl.pallas_call(
    _post_mm_body,
    grid=(GRID,),
    in_specs=[_pblk, _full, _col, _col, _col, _col, _bias],
    out_specs=_rowblk,
    out_shape=_out_t,
)

_post = pl.pallas_call(
    _post_body,
    grid=(GRID,),
    in_specs=[_pblk, _col, _col, _bias],
    out_specs=_rowblk,
    out_shape=_out_t,
)


def kernel(edge_index, x, W1, b1, W2, b2):
    src = edge_index[0]
    dst = edge_index[1]
    pad = N + (lax.iota(jnp.int32, NW * TPC * CHUNK - E) % (N_PAD - N))
    main = jnp.concatenate([src, pad]).reshape(NW, TPC, CHUNK)
    maind = jnp.concatenate([dst, pad]).reshape(NW, TPC, CHUNK)
    padc = (N + (lax.iota(jnp.int32, NW * (TPCA - TPC) * CHUNK) % (N_PAD - N))).reshape(
        NW, TPCA - TPC, CHUNK)
    src3 = jnp.concatenate([main, padc], axis=1)
    dst3 = jnp.concatenate([maind, padc], axis=1)
    xp = jnp.pad(x, ((0, N_PAD - N), (0, 0)))
    z2 = jnp.zeros((2, N_PAD), jnp.float32)
    zN = jnp.zeros((N_PAD, D), jnp.float32)

    _deg, _agg = _sc_kernels()
    degs = _deg(src3, dst3, z2)
    dout0 = degs[0, 0].reshape(N_PAD, 1)
    dout1 = degs[1, 0].reshape(N_PAD, 1)
    din0 = degs[0, 1].reshape(N_PAD, 1)
    din1 = degs[1, 1].reshape(N_PAD, 1)
    b1r = b1.reshape(1, D)
    b2r = b2.reshape(1, D)

    hs1 = _mm_scale(xp, W1, dout0, dout1)
    p1 = _agg(src3, dst3, hs1, zN)
    hs2 = _post_mm(p1, W2, din0, din1, dout0, dout1, b1r)
    p2 = _agg(src3, dst3, hs2, zN)
    out = _post(p2, din0, din1, b2r)
    return out[:N]


# speedup vs baseline: 1.4038x; 1.0838x over previous
import functools

import numpy as np

import jax
import jax.numpy as jnp
from jax import lax
from jax.experimental import pallas as pl
from jax.experimental.pallas import tpu as pltpu
from jax.experimental.pallas import tpu_sc as plsc

N = 10000
D = 128
NC = 2
NS = 16
NW = NC * NS
CHUNK = 120
N_PAD = 10240
E = 320000
TPC = 84
TPCA = TPC + 4
NTOT = NW * TPCA
SEG = N_PAD // NS

_TAIL = (N + (np.arange(NTOT * CHUNK - E, dtype=np.int32) % (N_PAD - N)))


@functools.cache
def _sc_kernels():
    mesh = plsc.VectorSubcoreMesh(
        core_axis_name="c", subcore_axis_name="s", num_cores=NC, num_subcores=NS
    )
    deg = pl.kernel(
        _deg_body,
        out_type=jax.ShapeDtypeStruct((NC, 2, N_PAD), jnp.float32),
        mesh=mesh,
        scratch_types=[
            pltpu.VMEM((TPCA, CHUNK), jnp.int32),
            pltpu.VMEM((TPCA, CHUNK), jnp.int32),
            pltpu.VMEM((CHUNK,), jnp.float32),
            pltpu.VMEM((SEG,), jnp.float32),
            pltpu.VMEM_SHARED((N_PAD,), jnp.float32),
            pltpu.VMEM_SHARED((N_PAD,), jnp.float32),
            pltpu.SemaphoreType.DMA,
        ],
    )
    agg = pl.kernel(
        _agg_body,
        out_type=jax.ShapeDtypeStruct((NC, N_PAD, D), jnp.float32),
        mesh=mesh,
        scratch_types=(
            [pltpu.VMEM((CHUNK,), jnp.int32)] * 6
            + [pltpu.VMEM((CHUNK, D), jnp.float32)] * 3
            + [pltpu.VMEM_SHARED((N_PAD, D), jnp.float32)]
            + [pltpu.SemaphoreType.DMA] * 12
        ),
    )
    return deg, agg


def _deg_body(src_hbm, dst_hbm, out_hbm, sidx, didx, ones_v, zbuf,
              dout_acc, din_acc, dsem):
    c = lax.axis_index("c")
    s = lax.axis_index("s")
    wid = s * NC + c

    def _zf(i, carry):
        zbuf[pl.ds(i * 16, 16)] = jnp.zeros((16,), jnp.float32)
        return carry

    lax.fori_loop(0, SEG // 16, _zf, 0)
    pltpu.sync_copy(zbuf, dout_acc.at[pl.ds(s * SEG, SEG)])
    pltpu.sync_copy(zbuf, din_acc.at[pl.ds(s * SEG, SEG)])
    pltpu.sync_copy(src_hbm.at[pl.ds(wid * TPCA, TPCA)], sidx)
    pltpu.sync_copy(dst_hbm.at[pl.ds(wid * TPCA, TPCA)], didx)
    for off in (0, 16, 32, 48, 64, 80, 96, CHUNK - 16):
        ones_v[pl.ds(off, 16)] = jnp.ones((16,), jnp.float32)
    plsc.subcore_barrier()

    def _step(j, carry):
        pltpu.async_copy(ones_v, dout_acc.at[sidx.at[j]], dsem, add=True)
        pltpu.async_copy(ones_v, din_acc.at[didx.at[j]], dsem, add=True)
        return carry

    lax.fori_loop(0, TPCA, _step, 0)

    def _drain(j, carry):
        pltpu.make_async_copy(ones_v, dout_acc.at[sidx.at[0]], dsem).wait()
        pltpu.make_async_copy(ones_v, din_acc.at[didx.at[0]], dsem).wait()
        return carry

    lax.fori_loop(0, TPCA, _drain, 0)
    plsc.subcore_barrier()
    pltpu.sync_copy(dout_acc.at[pl.ds(s * SEG, SEG)], out_hbm.at[c, 0, pl.ds(s * SEG, SEG)])
    pltpu.sync_copy(din_acc.at[pl.ds(s * SEG, SEG)], out_hbm.at[c, 1, pl.ds(s * SEG, SEG)])


def _agg_body(src_hbm, dst_hbm, hs_hbm, out_hbm,
              sr0, sr1, sr2, dr0, dr1, dr2, rows0, rows1, rows2, acc,
              g0, g1, g2, s0, s1, s2, i0, i1, i2, d0, d1, d2):
    c = lax.axis_index("c")
    s = lax.axis_index("s")
    wid = s * NC + c
    srs = (sr0, sr1, sr2)
    drs = (dr0, dr1, dr2)
    rows = (rows0, rows1, rows2)
    gs = (g0, g1, g2)
    ss = (s0, s1, s2)
    is_ = (i0, i1, i2)
    ds_ = (d0, d1, d2)

    def _zf(i, carry):
        for off in (0, 16, 32, 48, 64, 80, 96, D - 16):
            rows0[i, pl.ds(off, 16)] = jnp.zeros((16,), jnp.float32)
        return carry

    lax.fori_loop(0, CHUNK, _zf, 0)
    for k in range(SEG // CHUNK):
        pltpu.sync_copy(rows0, acc.at[pl.ds(s * SEG + k * CHUNK, CHUNK)])
    _rem = SEG - (SEG // CHUNK) * CHUNK
    pltpu.sync_copy(rows0.at[pl.ds(0, _rem)],
                    acc.at[pl.ds(s * SEG + SEG - _rem, _rem)])

    def ifetch(j, b):
        pltpu.async_copy(src_hbm.at[wid + NW * j], srs[b], is_[b])

    def iwait(b):
        pltpu.make_async_copy(src_hbm.at[0], srs[b], is_[b]).wait()

    def dfetch(j, b):
        pltpu.async_copy(dst_hbm.at[wid + NW * j], drs[b], ds_[b])

    def dwait(b):
        pltpu.make_async_copy(dst_hbm.at[0], drs[b], ds_[b]).wait()

    def gather(b):
        pltpu.async_copy(hs_hbm.at[srs[b]], rows[b], gs[b])

    def gwait(b):
        pltpu.make_async_copy(hs_hbm.at[sr0], rows[b], gs[b]).wait()

    def scat(b):
        pltpu.async_copy(rows[b], acc.at[drs[b]], ss[b], add=True)

    def swait(b):
        pltpu.make_async_copy(rows[b], acc.at[dr0], ss[b]).wait()

    ifetch(0, 0)
    ifetch(1, 1)
    ifetch(2, 2)
    dfetch(0, 0)
    dfetch(1, 1)
    iwait(0)
    gather(0)
    iwait(1)
    gather(1)
    plsc.subcore_barrier()

    def _stage(j, b, guard):
        cs = (b + 2) % 3
        gwait(b)
        dwait(b)
        scat(b)
        ifetch(j + 3, b)

        if guard is None:
            swait(cs)
        else:
            @pl.when(guard)
            def _():
                swait(cs)

        dfetch(j + 2, cs)
        iwait(cs)
        gather(cs)

    def _body(i, carry):
        j0 = 3 * i
        _stage(j0, 0, i > 0)
        _stage(j0 + 1, 1, None)
        _stage(j0 + 2, 2, None)
        return carry

    lax.fori_loop(0, TPC // 3, _body, 0)
    gwait(0)
    gwait(1)
    swait(2)
    iwait(2)
    dwait(0)
    dwait(1)
    plsc.subcore_barrier()
    pltpu.sync_copy(acc.at[pl.ds(s * SEG, SEG)], out_hbm.at[c, pl.ds(s * SEG, SEG)])


NREST = N_PAD - N


def _nrm(d_ref):
    return lax.rsqrt(jnp.maximum(d_ref[...], 1.0))


def _b1_body(x_ref, w_ref, ds_ref, o_ref):
    mm = jnp.dot(x_ref[...], w_ref[...], preferred_element_type=jnp.float32)
    o_ref[pl.ds(0, N), :] = mm * _nrm(ds_ref)
    o_ref[pl.ds(N, NREST), :] = jnp.zeros((NREST, D), jnp.float32)


def _b2_body(p_ref, w_ref, di_ref, ds_ref, b_ref, o_ref):
    p = (p_ref[0] + p_ref[1])[:N]
    h = jnp.maximum(p * _nrm(di_ref) + b_ref[...], 0.0)
    mm = jnp.dot(h, w_ref[...], preferred_element_type=jnp.float32)
    o_ref[pl.ds(0, N), :] = mm * _nrm(ds_ref)
    o_ref[pl.ds(N, NREST), :] = jnp.zeros((NREST, D), jnp.float32)


def _b3_body(p_ref, di_ref, b_ref, o_ref):
    p = (p_ref[0] + p_ref[1])[:N]
    o_ref[...] = jnp.maximum(p * _nrm(di_ref) + b_ref[...], 0.0)


_b1 = pl.pallas_call(_b1_body, out_shape=jax.ShapeDtypeStruct((N_PAD, D), jnp.float32))
_b2 = pl.pallas_call(_b2_body, out_shape=jax.ShapeDtypeStruct((N_PAD, D), jnp.float32))
_b3 = pl.pallas_call(_b3_body, out_shape=jax.ShapeDtypeStruct((N, D), jnp.float32))


def kernel(edge_index, x, W1, b1, W2, b2):
    src = edge_index[0]
    dst = edge_index[1]
    tail = jnp.asarray(_TAIL)
    srcc = jnp.concatenate([src, tail]).reshape(NTOT, CHUNK)
    dstc = jnp.concatenate([dst, tail]).reshape(NTOT, CHUNK)

    _deg, _agg = _sc_kernels()
    degs = _deg(srcc, dstc)
    dsum = degs[0] + degs[1]
    dsrc = dsum[0, :N].reshape(N, 1)
    din = dsum[1, :N].reshape(N, 1)

    hs1 = _b1(x, W1, dsrc)
    p1 = _agg(srcc, dstc, hs1)
    hs2 = _b2(p1, W2, din, dsrc, b1.reshape(1, D))
    p2 = _agg(srcc, dstc, hs2)
    return _b3(p2, din, b2.reshape(1, D))


# speedup vs baseline: 1.4322x; 1.0202x over previous
import functools

import numpy as np

import jax
import jax.numpy as jnp
from jax import lax
from jax.experimental import pallas as pl
from jax.experimental.pallas import tpu as pltpu
from jax.experimental.pallas import tpu_sc as plsc

N = 10000
D = 128
NC = 2
NS = 16
NW = NC * NS
CHUNK = 120
N_PAD = 10240
E = 320000
TPC = 84
TPCA = TPC + 4
NTOT = NW * TPCA
SEG = N_PAD // NS

_TAIL = (N + (np.arange(NTOT * CHUNK - E, dtype=np.int32) % (N_PAD - N)))


@functools.cache
def _sc_kernels():
    mesh = plsc.VectorSubcoreMesh(
        core_axis_name="c", subcore_axis_name="s", num_cores=NC, num_subcores=NS
    )
    deg = pl.kernel(
        _deg_body,
        out_type=jax.ShapeDtypeStruct((NC, 2, N_PAD), jnp.float32),
        mesh=mesh,
        scratch_types=[
            pltpu.VMEM((TPCA, CHUNK), jnp.int32),
            pltpu.VMEM((TPCA, CHUNK), jnp.int32),
            pltpu.VMEM((CHUNK,), jnp.float32),
            pltpu.VMEM((SEG,), jnp.float32),
            pltpu.VMEM_SHARED((N_PAD,), jnp.float32),
            pltpu.VMEM_SHARED((N_PAD,), jnp.float32),
            pltpu.SemaphoreType.DMA,
        ],
    )
    agg = pl.kernel(
        _agg_body,
        out_type=jax.ShapeDtypeStruct((NC, N_PAD, D), jnp.float32),
        mesh=mesh,
        scratch_types=(
            [pltpu.VMEM((CHUNK,), jnp.int32)] * 6
            + [pltpu.VMEM((CHUNK, D), jnp.float32)] * 3
            + [pltpu.VMEM_SHARED((N_PAD, D), jnp.float32)]
            + [pltpu.SemaphoreType.DMA] * 12
        ),
    )
    return deg, agg


def _deg_body(src_hbm, dst_hbm, out_hbm, sidx, didx, ones_v, zbuf,
              dout_acc, din_acc, dsem):
    c = lax.axis_index("c")
    s = lax.axis_index("s")
    wid = s * NC + c

    def _zf(i, carry):
        zbuf[pl.ds(i * 16, 16)] = jnp.zeros((16,), jnp.float32)
        return carry

    lax.fori_loop(0, SEG // 16, _zf, 0)
    pltpu.sync_copy(zbuf, dout_acc.at[pl.ds(s * SEG, SEG)])
    pltpu.sync_copy(zbuf, din_acc.at[pl.ds(s * SEG, SEG)])
    def _sf(j, carry):
        base = (wid * TPCA + j) * CHUNK
        pltpu.async_copy(src_hbm.at[pl.ds(base, CHUNK)], sidx.at[j], dsem)
        pltpu.async_copy(dst_hbm.at[pl.ds(base, CHUNK)], didx.at[j], dsem)
        return carry

    lax.fori_loop(0, TPCA, _sf, 0)

    def _sd(j, carry):
        pltpu.make_async_copy(src_hbm.at[pl.ds(0, CHUNK)], sidx.at[0], dsem).wait()
        pltpu.make_async_copy(dst_hbm.at[pl.ds(0, CHUNK)], didx.at[0], dsem).wait()
        return carry

    lax.fori_loop(0, TPCA, _sd, 0)
    for off in (0, 16, 32, 48, 64, 80, 96, CHUNK - 16):
        ones_v[pl.ds(off, 16)] = jnp.ones((16,), jnp.float32)
    plsc.subcore_barrier()

    def _step(j, carry):
        pltpu.async_copy(ones_v, dout_acc.at[sidx.at[j]], dsem, add=True)
        pltpu.async_copy(ones_v, din_acc.at[didx.at[j]], dsem, add=True)
        return carry

    lax.fori_loop(0, TPCA, _step, 0)

    def _drain(j, carry):
        pltpu.make_async_copy(ones_v, dout_acc.at[sidx.at[0]], dsem).wait()
        pltpu.make_async_copy(ones_v, din_acc.at[didx.at[0]], dsem).wait()
        return carry

    lax.fori_loop(0, TPCA, _drain, 0)
    plsc.subcore_barrier()
    pltpu.sync_copy(dout_acc.at[pl.ds(s * SEG, SEG)], out_hbm.at[c, 0, pl.ds(s * SEG, SEG)])
    pltpu.sync_copy(din_acc.at[pl.ds(s * SEG, SEG)], out_hbm.at[c, 1, pl.ds(s * SEG, SEG)])


def _agg_body(src_hbm, dst_hbm, hs_hbm, out_hbm,
              sr0, sr1, sr2, dr0, dr1, dr2, rows0, rows1, rows2, acc,
              g0, g1, g2, s0, s1, s2, i0, i1, i2, d0, d1, d2):
    c = lax.axis_index("c")
    s = lax.axis_index("s")
    wid = s * NC + c
    srs = (sr0, sr1, sr2)
    drs = (dr0, dr1, dr2)
    rows = (rows0, rows1, rows2)
    gs = (g0, g1, g2)
    ss = (s0, s1, s2)
    is_ = (i0, i1, i2)
    ds_ = (d0, d1, d2)

    def _zf(i, carry):
        for off in (0, 16, 32, 48, 64, 80, 96, D - 16):
            rows0[i, pl.ds(off, 16)] = jnp.zeros((16,), jnp.float32)
        return carry

    lax.fori_loop(0, CHUNK, _zf, 0)
    for k in range(SEG // CHUNK):
        pltpu.sync_copy(rows0, acc.at[pl.ds(s * SEG + k * CHUNK, CHUNK)])
    _rem = SEG - (SEG // CHUNK) * CHUNK
    pltpu.sync_copy(rows0.at[pl.ds(0, _rem)],
                    acc.at[pl.ds(s * SEG + SEG - _rem, _rem)])

    def ifetch(j, b):
        pltpu.async_copy(src_hbm.at[pl.ds((wid + NW * j) * CHUNK, CHUNK)],
                         srs[b], is_[b])

    def iwait(b):
        pltpu.make_async_copy(src_hbm.at[pl.ds(0, CHUNK)], srs[b], is_[b]).wait()

    def dfetch(j, b):
        pltpu.async_copy(dst_hbm.at[pl.ds((wid + NW * j) * CHUNK, CHUNK)],
                         drs[b], ds_[b])

    def dwait(b):
        pltpu.make_async_copy(dst_hbm.at[pl.ds(0, CHUNK)], drs[b], ds_[b]).wait()

    def gather(b):
        pltpu.async_copy(hs_hbm.at[srs[b]], rows[b], gs[b])

    def gwait(b):
        pltpu.make_async_copy(hs_hbm.at[sr0], rows[b], gs[b]).wait()

    def scat(b):
        pltpu.async_copy(rows[b], acc.at[drs[b]], ss[b], add=True)

    def swait(b):
        pltpu.make_async_copy(rows[b], acc.at[dr0], ss[b]).wait()

    ifetch(0, 0)
    ifetch(1, 1)
    ifetch(2, 2)
    dfetch(0, 0)
    dfetch(1, 1)
    iwait(0)
    gather(0)
    iwait(1)
    gather(1)
    plsc.subcore_barrier()

    def _stage(j, b, guard):
        cs = (b + 2) % 3
        gwait(b)
        dwait(b)
        scat(b)
        ifetch(j + 3, b)

        if guard is None:
            swait(cs)
        else:
            @pl.when(guard)
            def _():
                swait(cs)

        dfetch(j + 2, cs)
        iwait(cs)
        gather(cs)

    def _body(i, carry):
        j0 = 3 * i
        _stage(j0, 0, i > 0)
        _stage(j0 + 1, 1, None)
        _stage(j0 + 2, 2, None)
        return carry

    lax.fori_loop(0, TPC // 3, _body, 0)
    gwait(0)
    gwait(1)
    swait(2)
    iwait(2)
    dwait(0)
    dwait(1)
    plsc.subcore_barrier()
    pltpu.sync_copy(acc.at[pl.ds(s * SEG, SEG)], out_hbm.at[c, pl.ds(s * SEG, SEG)])


NREST = N_PAD - N


def _nrm(d_ref):
    return lax.rsqrt(jnp.maximum(d_ref[...], 1.0))


def _b1_body(x_ref, w_ref, ds_ref, o_ref):
    mm = jnp.dot(x_ref[...], w_ref[...], preferred_element_type=jnp.float32)
    o_ref[pl.ds(0, N), :] = mm * _nrm(ds_ref)
    o_ref[pl.ds(N, NREST), :] = jnp.zeros((NREST, D), jnp.float32)


def _b2_body(p_ref, w_ref, di_ref, ds_ref, b_ref, o_ref):
    p = (p_ref[0] + p_ref[1])[:N]
    h = jnp.maximum(p * _nrm(di_ref) + b_ref[...], 0.0)
    mm = jnp.dot(h, w_ref[...], preferred_element_type=jnp.float32)
    o_ref[pl.ds(0, N), :] = mm * _nrm(ds_ref)
    o_ref[pl.ds(N, NREST), :] = jnp.zeros((NREST, D), jnp.float32)


def _b3_body(p_ref, di_ref, b_ref, o_ref):
    p = (p_ref[0] + p_ref[1])[:N]
    o_ref[...] = jnp.maximum(p * _nrm(di_ref) + b_ref[...], 0.0)


_b1 = pl.pallas_call(_b1_body, out_shape=jax.ShapeDtypeStruct((N_PAD, D), jnp.float32))
_b2 = pl.pallas_call(_b2_body, out_shape=jax.ShapeDtypeStruct((N_PAD, D), jnp.float32))
_b3 = pl.pallas_call(_b3_body, out_shape=jax.ShapeDtypeStruct((N, D), jnp.float32))


def kernel(edge_index, x, W1, b1, W2, b2):
    src = edge_index[0]
    dst = edge_index[1]
    tail = jnp.asarray(_TAIL)
    srcc = jnp.concatenate([src, tail])
    dstc = jnp.concatenate([dst, tail])

    _deg, _agg = _sc_kernels()
    degs = _deg(srcc, dstc)
    dsum = degs[0] + degs[1]
    dsrc = dsum[0, :N].reshape(N, 1)
    din = dsum[1, :N].reshape(N, 1)

    hs1 = _b1(x, W1, dsrc)
    p1 = _agg(srcc, dstc, hs1)
    hs2 = _b2(p1, W2, din, dsrc, b1.reshape(1, D))
    p2 = _agg(srcc, dstc, hs2)
    return _b3(p2, din, b2.reshape(1, D))


# speedup vs baseline: 1.4382x; 1.0042x over previous
import functools

import numpy as np

import jax
import jax.numpy as jnp
from jax import lax
from jax.experimental import pallas as pl
from jax.experimental.pallas import tpu as pltpu
from jax.experimental.pallas import tpu_sc as plsc

N = 10000
D = 128
NC = 2
NS = 16
NW = NC * NS
CHUNK = 120
N_PAD = 10240
E = 320000
TPC = 84
TPCA = TPC + 4
NTOT = NW * TPCA
SEG = N_PAD // NS

_TAIL = (N + (np.arange(NTOT * CHUNK - E, dtype=np.int32) % (N_PAD - N)))


@functools.cache
def _sc_kernels():
    mesh = plsc.VectorSubcoreMesh(
        core_axis_name="c", subcore_axis_name="s", num_cores=NC, num_subcores=NS
    )
    deg = pl.kernel(
        _deg_body,
        out_type=jax.ShapeDtypeStruct((NC, 2, N_PAD), jnp.float32),
        mesh=mesh,
        scratch_types=[
            pltpu.VMEM((TPCA, CHUNK), jnp.int32),
            pltpu.VMEM((TPCA, CHUNK), jnp.int32),
            pltpu.VMEM((CHUNK,), jnp.float32),
            pltpu.VMEM((SEG,), jnp.float32),
            pltpu.VMEM_SHARED((N_PAD,), jnp.float32),
            pltpu.VMEM_SHARED((N_PAD,), jnp.float32),
            pltpu.SemaphoreType.DMA,
        ],
    )
    agg = pl.kernel(
        _agg_body,
        out_type=jax.ShapeDtypeStruct((NC, N_PAD, D), jnp.float32),
        mesh=mesh,
        scratch_types=(
            [pltpu.VMEM((CHUNK,), jnp.int32)] * 6
            + [pltpu.VMEM((CHUNK, D), jnp.float32)] * 3
            + [pltpu.VMEM_SHARED((N_PAD, D), jnp.float32)]
            + [pltpu.SemaphoreType.DMA] * 12
        ),
    )
    return deg, agg


def _deg_body(src_hbm, dst_hbm, out_hbm, sidx, didx, ones_v, zbuf,
              dout_acc, din_acc, dsem):
    c = lax.axis_index("c")
    s = lax.axis_index("s")
    wid = s * NC + c

    def _zf(i, carry):
        zbuf[pl.ds(i * 16, 16)] = jnp.zeros((16,), jnp.float32)
        return carry

    lax.fori_loop(0, SEG // 16, _zf, 0)
    pltpu.sync_copy(zbuf, dout_acc.at[pl.ds(s * SEG, SEG)])
    pltpu.sync_copy(zbuf, din_acc.at[pl.ds(s * SEG, SEG)])
    def _sf(j, carry):
        base = (wid * TPCA + j) * CHUNK
        pltpu.async_copy(src_hbm.at[pl.ds(base, CHUNK)], sidx.at[j], dsem)
        pltpu.async_copy(dst_hbm.at[pl.ds(base, CHUNK)], didx.at[j], dsem)
        return carry

    lax.fori_loop(0, TPCA, _sf, 0)

    def _sd(j, carry):
        pltpu.make_async_copy(src_hbm.at[pl.ds(0, CHUNK)], sidx.at[0], dsem).wait()
        pltpu.make_async_copy(dst_hbm.at[pl.ds(0, CHUNK)], didx.at[0], dsem).wait()
        return carry

    lax.fori_loop(0, TPCA, _sd, 0)
    for off in (0, 16, 32, 48, 64, 80, 96, CHUNK - 16):
        ones_v[pl.ds(off, 16)] = jnp.ones((16,), jnp.float32)
    plsc.subcore_barrier()

    def _step(j, carry):
        pltpu.async_copy(ones_v, dout_acc.at[sidx.at[j]], dsem, add=True)
        pltpu.async_copy(ones_v, din_acc.at[didx.at[j]], dsem, add=True)
        return carry

    lax.fori_loop(0, TPCA, _step, 0)

    def _drain(j, carry):
        pltpu.make_async_copy(ones_v, dout_acc.at[sidx.at[0]], dsem).wait()
        pltpu.make_async_copy(ones_v, din_acc.at[didx.at[0]], dsem).wait()
        return carry

    lax.fori_loop(0, TPCA, _drain, 0)
    plsc.subcore_barrier()
    pltpu.sync_copy(dout_acc.at[pl.ds(s * SEG, SEG)], out_hbm.at[c, 0, pl.ds(s * SEG, SEG)])
    pltpu.sync_copy(din_acc.at[pl.ds(s * SEG, SEG)], out_hbm.at[c, 1, pl.ds(s * SEG, SEG)])


def _agg_body(src_hbm, dst_hbm, hs_hbm, out_hbm,
              sr0, sr1, sr2, dr0, dr1, dr2, rows0, rows1, rows2, acc,
              g0, g1, g2, s0, s1, s2, i0, i1, i2, d0, d1, d2):
    c = lax.axis_index("c")
    s = lax.axis_index("s")
    wid = s * NC + c
    srs = (sr0, sr1, sr2)
    drs = (dr0, dr1, dr2)
    rows = (rows0, rows1, rows2)
    gs = (g0, g1, g2)
    ss = (s0, s1, s2)
    is_ = (i0, i1, i2)
    ds_ = (d0, d1, d2)

    def ifetch(j, b):
        pltpu.async_copy(src_hbm.at[pl.ds((wid + NW * j) * CHUNK, CHUNK)],
                         srs[b], is_[b])

    def iwait(b):
        pltpu.make_async_copy(src_hbm.at[pl.ds(0, CHUNK)], srs[b], is_[b]).wait()

    def dfetch(j, b):
        pltpu.async_copy(dst_hbm.at[pl.ds((wid + NW * j) * CHUNK, CHUNK)],
                         drs[b], ds_[b])

    def dwait(b):
        pltpu.make_async_copy(dst_hbm.at[pl.ds(0, CHUNK)], drs[b], ds_[b]).wait()

    def gather(b):
        pltpu.async_copy(hs_hbm.at[srs[b]], rows[b], gs[b])

    def gwait(b):
        pltpu.make_async_copy(hs_hbm.at[sr0], rows[b], gs[b]).wait()

    def scat(b):
        pltpu.async_copy(rows[b], acc.at[drs[b]], ss[b], add=True)

    def swait(b):
        pltpu.make_async_copy(rows[b], acc.at[dr0], ss[b]).wait()

    ifetch(0, 0)
    ifetch(1, 1)
    ifetch(2, 2)
    dfetch(0, 0)
    dfetch(1, 1)

    def _zf(i, carry):
        for off in (0, 16, 32, 48, 64, 80, 96, D - 16):
            rows2[i, pl.ds(off, 16)] = jnp.zeros((16,), jnp.float32)
        return carry

    lax.fori_loop(0, CHUNK, _zf, 0)
    for k in range(SEG // CHUNK):
        pltpu.sync_copy(rows2, acc.at[pl.ds(s * SEG + k * CHUNK, CHUNK)])
    _rem = SEG - (SEG // CHUNK) * CHUNK
    pltpu.sync_copy(rows2.at[pl.ds(0, _rem)],
                    acc.at[pl.ds(s * SEG + SEG - _rem, _rem)])

    iwait(0)
    gather(0)
    iwait(1)
    gather(1)
    plsc.subcore_barrier()

    def _stage(j, b, guard):
        cs = (b + 2) % 3
        gwait(b)
        dwait(b)
        scat(b)
        ifetch(j + 3, b)

        if guard is None:
            swait(cs)
        else:
            @pl.when(guard)
            def _():
                swait(cs)

        dfetch(j + 2, cs)
        iwait(cs)
        gather(cs)

    def _body(i, carry):
        j0 = 3 * i
        _stage(j0, 0, i > 0)
        _stage(j0 + 1, 1, None)
        _stage(j0 + 2, 2, None)
        return carry

    lax.fori_loop(0, TPC // 3, _body, 0)
    gwait(0)
    gwait(1)
    swait(2)
    iwait(2)
    dwait(0)
    dwait(1)
    plsc.subcore_barrier()
    pltpu.sync_copy(acc.at[pl.ds(s * SEG, SEG)], out_hbm.at[c, pl.ds(s * SEG, SEG)])


NREST = N_PAD - N


def _nrm(d_ref):
    return lax.rsqrt(jnp.maximum(d_ref[...], 1.0))


def _m1_body(x_ref, w_ref, o_ref):
    o_ref[...] = jnp.dot(x_ref[...], w_ref[...], preferred_element_type=jnp.float32)


def _b1_body(m_ref, ds_ref, o_ref):
    o_ref[pl.ds(0, N), :] = m_ref[...] * _nrm(ds_ref)
    o_ref[pl.ds(N, NREST), :] = jnp.zeros((NREST, D), jnp.float32)


def _b2_body(p_ref, w_ref, di_ref, ds_ref, b_ref, o_ref):
    p = (p_ref[0] + p_ref[1])[:N]
    h = jnp.maximum(p * _nrm(di_ref) + b_ref[...], 0.0)
    mm = jnp.dot(h, w_ref[...], preferred_element_type=jnp.float32)
    o_ref[pl.ds(0, N), :] = mm * _nrm(ds_ref)
    o_ref[pl.ds(N, NREST), :] = jnp.zeros((NREST, D), jnp.float32)


def _b3_body(p_ref, di_ref, b_ref, o_ref):
    p = (p_ref[0] + p_ref[1])[:N]
    o_ref[...] = jnp.maximum(p * _nrm(di_ref) + b_ref[...], 0.0)


_m1 = pl.pallas_call(_m1_body, out_shape=jax.ShapeDtypeStruct((N, D), jnp.float32))
_b1 = pl.pallas_call(_b1_body, out_shape=jax.ShapeDtypeStruct((N_PAD, D), jnp.float32))
_b2 = pl.pallas_call(_b2_body, out_shape=jax.ShapeDtypeStruct((N_PAD, D), jnp.float32))
_b3 = pl.pallas_call(_b3_body, out_shape=jax.ShapeDtypeStruct((N, D), jnp.float32))


def kernel(edge_index, x, W1, b1, W2, b2):
    src = edge_index[0]
    dst = edge_index[1]
    tail = jnp.asarray(_TAIL)
    srcc = jnp.concatenate([src, tail])
    dstc = jnp.concatenate([dst, tail])

    _deg, _agg = _sc_kernels()
    m1 = _m1(x, W1)
    degs = _deg(srcc, dstc)
    dsum = degs[0] + degs[1]
    dsrc = dsum[0, :N].reshape(N, 1)
    din = dsum[1, :N].reshape(N, 1)

    hs1 = _b1(m1, dsrc)
    p1 = _agg(srcc, dstc, hs1)
    hs2 = _b2(p1, W2, din, dsrc, b1.reshape(1, D))
    p2 = _agg(srcc, dstc, hs2)
    return _b3(p2, din, b2.reshape(1, D))
